# baseline (speedup 1.0000x reference)
import jax
import jax.numpy as jnp
from jax import lax
from jax.experimental import pallas as pl
from jax.experimental.pallas import tpu as pltpu, tpu_sc as plsc

_INPUT_LEN = 200
_EMBED_DIM = 64
_BATCH = 4096
_FLAT = _INPUT_LEN * _EMBED_DIM

_NC = 2
_NS = 16
_NW = _NC * _NS
_ROWS_PER_W = _BATCH // _NW
_REP = 8
_BLOCKS_PER_W = _ROWS_PER_W // _REP


def _make_sc_kernel():
    mesh = plsc.VectorSubcoreMesh(core_axis_name="c", subcore_axis_name="s")

    @pl.kernel(
        mesh=mesh,
        out_type=jax.ShapeDtypeStruct((_BATCH, _FLAT), jnp.float32),
        scratch_types=[
            pltpu.VMEM((_REP, _FLAT), jnp.float32),
            pltpu.SemaphoreType.DMA,
        ],
    )
    def sc_kernel(pos_hbm, out_hbm, tile_v, sem):
        wid = lax.axis_index("s") * _NC + lax.axis_index("c")
        base = wid * _ROWS_PER_W
        fills = [pltpu.async_copy(pos_hbm, tile_v.at[r], sem) for r in range(_REP)]
        for f in fills:
            f.wait()
        outs = [
            pltpu.async_copy(
                tile_v, out_hbm.at[pl.ds(base + j * _REP, _REP), :], sem
            )
            for j in range(_BLOCKS_PER_W)
        ]
        for c in outs:
            c.wait()

    return sc_kernel


_SC_KERNEL = _make_sc_kernel()


def kernel(x, pos_table):
    del x
    pos_flat = pos_table.reshape(_FLAT)
    out = _SC_KERNEL(pos_flat)
    return out.reshape(_BATCH, _INPUT_LEN, _EMBED_DIM)
